# baseline (speedup 1.0000x reference)
import functools

import jax
import jax.numpy as jnp
from jax import lax
from jax.experimental import pallas as pl
from jax.experimental.pallas import tpu as pltpu
from jax.experimental.pallas import tpu_sc as plsc

N = 10000
NP = 10112
SP = NP // 16
E = 160000
K = 128
HM = 40
NMEGA = 2
NCH = HM * NMEGA
EP = 16 * NCH * K
DIM = 256
HALF = 128
BN1 = 1000
BN2 = 632


def _mm1_body(x_ref, w_ref, a_ref, f_ref, s_ref):
    feat = jnp.dot(x_ref[...], w_ref[...], preferred_element_type=jnp.float32)
    f_ref[0] = feat[:, :HALF]
    f_ref[1] = feat[:, HALF:]
    s_ref[...] = jnp.dot(feat, a_ref[...], preferred_element_type=jnp.float32)


def _mm1(x, w, a):
    return pl.pallas_call(
        _mm1_body,
        grid=(N // BN1,),
        in_specs=[
            pl.BlockSpec((BN1, DIM), lambda i: (i, 0)),
            pl.BlockSpec((DIM, DIM), lambda i: (0, 0)),
            pl.BlockSpec((DIM, 8), lambda i: (0, 0)),
        ],
        out_specs=[
            pl.BlockSpec((2, BN1, HALF), lambda i: (0, i, 0)),
            pl.BlockSpec((BN1, 8), lambda i: (i, 0)),
        ],
        out_shape=[
            jax.ShapeDtypeStruct((2, N, HALF), jnp.float32),
            jax.ShapeDtypeStruct((N, 8), jnp.float32),
        ],
    )(x, w, a)


def _mm2_body(xa_ref, xb_ref, wa_ref, wb_ref, a_ref, f_ref, s_ref):
    feat = jnp.dot(xa_ref[...], wa_ref[...], preferred_element_type=jnp.float32)
    feat += jnp.dot(xb_ref[...], wb_ref[...], preferred_element_type=jnp.float32)
    f_ref[0] = feat[:, :HALF]
    f_ref[1] = feat[:, HALF:]
    s_ref[...] = jnp.dot(feat, a_ref[...], preferred_element_type=jnp.float32)


def _mm2(xa, xb, wa, wb, a):
    return pl.pallas_call(
        _mm2_body,
        grid=(NP // BN2,),
        in_specs=[
            pl.BlockSpec((BN2, HALF), lambda i: (i, 0)),
            pl.BlockSpec((BN2, HALF), lambda i: (i, 0)),
            pl.BlockSpec((HALF, DIM), lambda i: (0, 0)),
            pl.BlockSpec((HALF, DIM), lambda i: (0, 0)),
            pl.BlockSpec((DIM, 8), lambda i: (0, 0)),
        ],
        out_specs=[
            pl.BlockSpec((2, BN2, HALF), lambda i: (0, i, 0)),
            pl.BlockSpec((BN2, 8), lambda i: (i, 0)),
        ],
        out_shape=[
            jax.ShapeDtypeStruct((2, NP, HALF), jnp.float32),
            jax.ShapeDtypeStruct((NP, 8), jnp.float32),
        ],
    )(xa, xb, wa, wb, a)


def _sc_body(elu, tabA, tabB, elh, erh, srcw, dstw, bias2, out,
             src_v, dst_v, r0, r1, els_b, erd_b, ee_b, den_b, bias_v, zv, idx_b,
             els2_b, erd2_b, ee2_b, idx2_b,
             el_s, er_s, den_s, acc_s, sem0, sem1,
             ssem0, ssem1, dsem0, dsem1, hsem0, hsem1):
    cid = lax.axis_index("c")
    sid = lax.axis_index("s")
    zero16 = jnp.zeros((16,), jnp.float32)
    base = sid * SP

    @pl.when(sid == 0)
    def _():
        pltpu.sync_copy(elh, el_s)

    @pl.when(sid == 1)
    def _():
        pltpu.sync_copy(erh, er_s)

    pltpu.sync_copy(bias2.at[cid], bias_v)

    def _z_row(r, carry):
        for j in range(8):
            r0[r, pl.ds(j * 16, 16)] = zero16
        return carry
    lax.fori_loop(0, K, _z_row, 0)
    for i in range(40):
        zv[pl.ds(i * 16, 16)] = zero16
    nfull = SP // K
    rem = SP - nfull * K
    for i in range(nfull):
        pltpu.sync_copy(r0, acc_s.at[pl.ds(base + i * K, K)])
    pltpu.sync_copy(r0.at[pl.ds(0, rem)], acc_s.at[pl.ds(base + nfull * K, rem)])
    pltpu.sync_copy(zv.at[pl.ds(0, SP)], den_s.at[pl.ds(base, SP)])
    plsc.subcore_barrier()

    rows = (r0, r1)
    elsb = (els_b, els2_b)
    erdb = (erd_b, erd2_b)
    eeb = (ee_b, ee2_b)
    idxb = (idx_b, idx2_b)
    gsem = (sem0, sem1)
    ssem = (ssem0, ssem1)
    dsem = (dsem0, dsem1)
    hsem = (hsem0, hsem1)

    def _g_row(c, buf, sem):
        @pl.when(cid == 0)
        def _():
            pltpu.async_copy(tabA.at[src_v.at[c]], buf, sem)

        @pl.when(cid == 1)
        def _():
            pltpu.async_copy(tabB.at[src_v.at[c]], buf, sem)

    def _g_small(c, b):
        pltpu.async_copy(el_s.at[src_v.at[c]], elsb[b], hsem[b])
        pltpu.async_copy(er_s.at[dst_v.at[c]], erdb[b], hsem[b])

    for m in range(NMEGA):
        pltpu.sync_copy(srcw.at[sid, pl.ds(m * HM, HM)], src_v)
        pltpu.sync_copy(dstw.at[sid, pl.ds(m * HM, HM)], dst_v)
        _g_small(0, 0)
        _g_small(1, 1)
        _g_row(0, r0, gsem[0])

        def _chunk(g, carry):
            for b in range(2):
                c = g * 2 + b
                ob = 1 - b

                @pl.when(c >= 2)
                def _():
                    pltpu.make_async_copy(
                        eeb[b], den_s.at[idxb[b]], dsem[b]).wait()

                pltpu.make_async_copy(
                    el_s.at[src_v.at[c]], elsb[b], hsem[b]).wait()
                pltpu.make_async_copy(
                    er_s.at[dst_v.at[c]], erdb[b], hsem[b]).wait()
                for j in range(K // 16):
                    sl = pl.ds(j * 16, 16)
                    d16 = dst_v[c, sl]
                    e = elsb[b][sl] + erdb[b][sl]
                    e = jnp.where(e > 0, e, 0.2 * e)
                    ee = jnp.exp(e)
                    eeb[b][sl] = jnp.where(d16 >= N, 0.0, ee)
                    idxb[b][sl] = d16
                pltpu.async_copy(eeb[b], den_s.at[idxb[b]], dsem[b], add=True)

                pltpu.make_async_copy(
                    tabA.at[src_v.at[c]], rows[b], gsem[b]).wait()

                def _scale(g2, carry2):
                    a16 = eeb[b][pl.ds(g2 * 16, 16)]
                    for i in range(16):
                        r = g2 * 16 + i
                        av = jnp.full((16,), a16[i], jnp.float32)
                        for j in range(8):
                            sl = pl.ds(j * 16, 16)
                            rows[b][r, sl] = rows[b][r, sl] * av
                    return carry2
                lax.fori_loop(0, K // 16, _scale, 0)
                pltpu.async_copy(rows[b], acc_s.at[idxb[b]], ssem[b], add=True)

                @pl.when(c >= 1)
                def _():
                    pltpu.make_async_copy(
                        rows[ob], acc_s.at[idxb[ob]], ssem[ob]).wait()

                @pl.when(c + 1 < HM)
                def _():
                    _g_row(c + 1, rows[ob], gsem[ob])

                @pl.when(c + 2 < HM)
                def _():
                    _g_small(c + 2, b)
            return carry
        lax.fori_loop(0, HM // 2, _chunk, 0)
        pltpu.make_async_copy(rows[1], acc_s.at[idxb[1]], ssem[1]).wait()
        pltpu.make_async_copy(eeb[0], den_s.at[idxb[0]], dsem[0]).wait()
        pltpu.make_async_copy(eeb[1], den_s.at[idxb[1]], dsem[1]).wait()
    plsc.subcore_barrier()

    def _norm(g2, carry):
        d16 = jnp.maximum(den_b[pl.ds(g2 * 16, 16)], 1e-9)
        inv16 = 1.0 / d16
        for i2 in range(16):
            r = g2 * 16 + i2
            dv = jnp.full((16,), inv16[i2], jnp.float32)
            for j in range(8):
                sl = pl.ds(j * 16, 16)
                v = r0[r, sl] * dv + bias_v[sl]
                if elu:
                    v = jnp.where(v > 0, v, jnp.exp(jnp.minimum(v, 0.0)) - 1.0)
                r0[r, sl] = v
        return carry

    def _copyout(cnt, i, carry):
        b0 = base + i * K
        pltpu.sync_copy(acc_s.at[pl.ds(b0, cnt)], r0.at[pl.ds(0, cnt)])
        pltpu.sync_copy(den_s.at[pl.ds(b0, cnt)], den_b.at[pl.ds(0, cnt)])
        lax.fori_loop(0, (cnt + 15) // 16, _norm, 0)
        pltpu.sync_copy(r0.at[pl.ds(0, cnt)], out.at[cid, pl.ds(b0, cnt)])
        return carry

    lax.fori_loop(0, nfull, functools.partial(_copyout, K), 0)
    _copyout(rem, nfull, 0)


def _make_sc(elu):
    mesh = plsc.VectorSubcoreMesh(core_axis_name="c", subcore_axis_name="s")
    return pl.kernel(
        functools.partial(_sc_body, elu),
        out_type=jax.ShapeDtypeStruct((2, NP, HALF), jnp.float32),
        mesh=mesh,
        compiler_params=pltpu.CompilerParams(needs_layout_passes=False),
        scratch_types=[
            pltpu.VMEM((HM, K), jnp.int32),
            pltpu.VMEM((HM, K), jnp.int32),
            pltpu.VMEM((K, HALF), jnp.float32),
            pltpu.VMEM((K, HALF), jnp.float32),
            pltpu.VMEM((K,), jnp.float32),
            pltpu.VMEM((K,), jnp.float32),
            pltpu.VMEM((K,), jnp.float32),
            pltpu.VMEM((K,), jnp.float32),
            pltpu.VMEM((HALF,), jnp.float32),
            pltpu.VMEM((640,), jnp.float32),
            pltpu.VMEM((K,), jnp.int32),
            pltpu.VMEM((K,), jnp.float32),
            pltpu.VMEM((K,), jnp.float32),
            pltpu.VMEM((K,), jnp.float32),
            pltpu.VMEM((K,), jnp.int32),
            pltpu.VMEM_SHARED((NP,), jnp.float32),
            pltpu.VMEM_SHARED((NP,), jnp.float32),
            pltpu.VMEM_SHARED((NP,), jnp.float32),
            pltpu.VMEM_SHARED((NP, HALF), jnp.float32),
            pltpu.SemaphoreType.DMA,
            pltpu.SemaphoreType.DMA,
            pltpu.SemaphoreType.DMA,
            pltpu.SemaphoreType.DMA,
            pltpu.SemaphoreType.DMA,
            pltpu.SemaphoreType.DMA,
            pltpu.SemaphoreType.DMA,
            pltpu.SemaphoreType.DMA,
        ],
    )


_sc_layer1 = _make_sc(True)
_sc_layer2 = _make_sc(False)


def _pack_attn(attn_l, attn_r):
    a = jnp.zeros((DIM, 8), jnp.float32)
    a = a.at[:, 0].set(attn_l)
    a = a.at[:, 1].set(attn_r)
    return a


@jax.jit
def kernel(features, edge_index, W1, attn_l1, attn_r1, b1, W2, attn_l2, attn_r2, b2):
    src = edge_index[0]
    dst = edge_index[1]
    pad = EP - E
    srcw = jnp.concatenate([src, jnp.zeros((pad,), jnp.int32)]).reshape(16, NCH, K)
    dstw = jnp.concatenate([dst, jnp.full((pad,), N, jnp.int32)]).reshape(16, NCH, K)

    f2, s = _mm1(features, W1, _pack_attn(attn_l1, attn_r1))
    b1s = jnp.stack([b1[:HALF], b1[HALF:]])
    zpad = jnp.zeros((NP - N,), jnp.float32)
    el1 = jnp.concatenate([s[:, 0], zpad])
    er1 = jnp.concatenate([s[:, 1], zpad])
    h = _sc_layer1(f2[0], f2[1], el1, er1, srcw, dstw, b1s)

    f2b, s2 = _mm2(h[0], h[1], W2[:HALF], W2[HALF:], _pack_attn(attn_l2, attn_r2))
    b2s = jnp.stack([b2[:HALF], b2[HALF:]])
    o = _sc_layer2(f2b[0], f2b[1], s2[:, 0], s2[:, 1], srcw, dstw, b2s)
    return jnp.concatenate([o[0, :N], o[1, :N]], axis=1)


# speedup vs baseline: 1.0013x; 1.0013x over previous
import functools

import jax
import jax.numpy as jnp
from jax import lax
from jax.experimental import pallas as pl
from jax.experimental.pallas import tpu as pltpu
from jax.experimental.pallas import tpu_sc as plsc

N = 10000
NP = 10112
SP = NP // 16
E = 160000
K = 128
HM = 40
NMEGA = 2
NCH = HM * NMEGA
EP = 16 * NCH * K
DIM = 256
HALF = 128
BN1 = 1000
BN2 = 632


def _mm1_body(x_ref, w_ref, a_ref, f_ref, s_ref):
    feat = jnp.dot(x_ref[...], w_ref[...], preferred_element_type=jnp.float32)
    f_ref[0] = feat[:, :HALF]
    f_ref[1] = feat[:, HALF:]
    s_ref[...] = jnp.dot(feat, a_ref[...], preferred_element_type=jnp.float32)


def _mm1(x, w, a):
    return pl.pallas_call(
        _mm1_body,
        grid=(N // BN1,),
        in_specs=[
            pl.BlockSpec((BN1, DIM), lambda i: (i, 0)),
            pl.BlockSpec((DIM, DIM), lambda i: (0, 0)),
            pl.BlockSpec((DIM, 8), lambda i: (0, 0)),
        ],
        out_specs=[
            pl.BlockSpec((2, BN1, HALF), lambda i: (0, i, 0)),
            pl.BlockSpec((BN1, 8), lambda i: (i, 0)),
        ],
        out_shape=[
            jax.ShapeDtypeStruct((2, N, HALF), jnp.float32),
            jax.ShapeDtypeStruct((N, 8), jnp.float32),
        ],
    )(x, w, a)


def _mm2_body(xa_ref, xb_ref, wa_ref, wb_ref, a_ref, f_ref, s_ref):
    feat = jnp.dot(xa_ref[...], wa_ref[...], preferred_element_type=jnp.float32)
    feat += jnp.dot(xb_ref[...], wb_ref[...], preferred_element_type=jnp.float32)
    f_ref[0] = feat[:, :HALF]
    f_ref[1] = feat[:, HALF:]
    s_ref[...] = jnp.dot(feat, a_ref[...], preferred_element_type=jnp.float32)


def _mm2(xa, xb, wa, wb, a):
    return pl.pallas_call(
        _mm2_body,
        grid=(NP // BN2,),
        in_specs=[
            pl.BlockSpec((BN2, HALF), lambda i: (i, 0)),
            pl.BlockSpec((BN2, HALF), lambda i: (i, 0)),
            pl.BlockSpec((HALF, DIM), lambda i: (0, 0)),
            pl.BlockSpec((HALF, DIM), lambda i: (0, 0)),
            pl.BlockSpec((DIM, 8), lambda i: (0, 0)),
        ],
        out_specs=[
            pl.BlockSpec((2, BN2, HALF), lambda i: (0, i, 0)),
            pl.BlockSpec((BN2, 8), lambda i: (i, 0)),
        ],
        out_shape=[
            jax.ShapeDtypeStruct((2, NP, HALF), jnp.float32),
            jax.ShapeDtypeStruct((NP, 8), jnp.float32),
        ],
    )(xa, xb, wa, wb, a)


def _sc_body(elu, tabA, tabB, elh, erh, srcw, dstw, bias2, out,
             src_v, dst_v, r0, r1, els_b, erd_b, ee_b, den_b, bias_v, zv, idx_b,
             els2_b, erd2_b, ee2_b, idx2_b,
             el_s, er_s, den_s, acc_s, sem0, sem1,
             ssem0, ssem1, dsem0, dsem1, hsem0, hsem1):
    cid = lax.axis_index("c")
    sid = lax.axis_index("s")
    zero16 = jnp.zeros((16,), jnp.float32)
    base = sid * SP

    @pl.when(sid == 0)
    def _():
        pltpu.sync_copy(elh, el_s)

    @pl.when(sid == 1)
    def _():
        pltpu.sync_copy(erh, er_s)

    pltpu.sync_copy(bias2.at[cid], bias_v)

    def _z_row(r, carry):
        for j in range(8):
            r0[r, pl.ds(j * 16, 16)] = zero16
        return carry
    lax.fori_loop(0, K, _z_row, 0)
    for i in range(40):
        zv[pl.ds(i * 16, 16)] = zero16
    nfull = SP // K
    rem = SP - nfull * K
    for i in range(nfull):
        pltpu.sync_copy(r0, acc_s.at[pl.ds(base + i * K, K)])
    pltpu.sync_copy(r0.at[pl.ds(0, rem)], acc_s.at[pl.ds(base + nfull * K, rem)])
    pltpu.sync_copy(zv.at[pl.ds(0, SP)], den_s.at[pl.ds(base, SP)])
    plsc.subcore_barrier()

    jax.named_scope
    rows = (r0, r1)
    elsb = (els_b, els2_b)
    erdb = (erd_b, erd2_b)
    eeb = (ee_b, ee2_b)
    idxb = (idx_b, idx2_b)
    gsem = (sem0, sem1)
    ssem = (ssem0, ssem1)
    dsem = (dsem0, dsem1)
    hsem = (hsem0, hsem1)

    def _g_row(c, buf, sem):
        @pl.when(cid == 0)
        def _():
            pltpu.async_copy(tabA.at[src_v.at[c]], buf, sem)

        @pl.when(cid == 1)
        def _():
            pltpu.async_copy(tabB.at[src_v.at[c]], buf, sem)

    def _g_small(c, b):
        pltpu.async_copy(el_s.at[src_v.at[c]], elsb[b], hsem[b])
        pltpu.async_copy(er_s.at[dst_v.at[c]], erdb[b], hsem[b])

    for m in range(NMEGA):
      with jax.named_scope("edge_mega"):
        pltpu.sync_copy(srcw.at[sid, pl.ds(m * HM, HM)], src_v)
        pltpu.sync_copy(dstw.at[sid, pl.ds(m * HM, HM)], dst_v)
        _g_small(0, 0)
        _g_small(1, 1)
        _g_row(0, r0, gsem[0])

        def _chunk(g, carry):
            for b in range(2):
                c = g * 2 + b
                ob = 1 - b

                @pl.when(c >= 2)
                def _():
                    pltpu.make_async_copy(
                        eeb[b], den_s.at[idxb[b]], dsem[b]).wait()

                pltpu.make_async_copy(
                    el_s.at[src_v.at[c]], elsb[b], hsem[b]).wait()
                pltpu.make_async_copy(
                    er_s.at[dst_v.at[c]], erdb[b], hsem[b]).wait()
                for j in range(K // 16):
                    sl = pl.ds(j * 16, 16)
                    d16 = dst_v[c, sl]
                    e = elsb[b][sl] + erdb[b][sl]
                    e = jnp.where(e > 0, e, 0.2 * e)
                    ee = jnp.exp(e)
                    eeb[b][sl] = jnp.where(d16 >= N, 0.0, ee)
                    idxb[b][sl] = d16
                pltpu.async_copy(eeb[b], den_s.at[idxb[b]], dsem[b], add=True)

                pltpu.make_async_copy(
                    tabA.at[src_v.at[c]], rows[b], gsem[b]).wait()

                def _scale(g2, carry2):
                    a16 = eeb[b][pl.ds(g2 * 16, 16)]
                    for i in range(16):
                        r = g2 * 16 + i
                        av = jnp.full((16,), a16[i], jnp.float32)
                        for j in range(8):
                            sl = pl.ds(j * 16, 16)
                            rows[b][r, sl] = rows[b][r, sl] * av
                    return carry2
                lax.fori_loop(0, K // 16, _scale, 0)
                pltpu.async_copy(rows[b], acc_s.at[idxb[b]], ssem[b], add=True)

                @pl.when(c >= 1)
                def _():
                    pltpu.make_async_copy(
                        rows[ob], acc_s.at[idxb[ob]], ssem[ob]).wait()

                @pl.when(c + 1 < HM)
                def _():
                    _g_row(c + 1, rows[ob], gsem[ob])

                @pl.when(c + 2 < HM)
                def _():
                    _g_small(c + 2, b)
            return carry
        lax.fori_loop(0, HM // 2, _chunk, 0)
        pltpu.make_async_copy(rows[1], acc_s.at[idxb[1]], ssem[1]).wait()
        pltpu.make_async_copy(eeb[0], den_s.at[idxb[0]], dsem[0]).wait()
        pltpu.make_async_copy(eeb[1], den_s.at[idxb[1]], dsem[1]).wait()
    plsc.subcore_barrier()

    def _norm(g2, carry):
        d16 = jnp.maximum(den_b[pl.ds(g2 * 16, 16)], 1e-9)
        inv16 = 1.0 / d16
        for i2 in range(16):
            r = g2 * 16 + i2
            dv = jnp.full((16,), inv16[i2], jnp.float32)
            for j in range(8):
                sl = pl.ds(j * 16, 16)
                v = r0[r, sl] * dv + bias_v[sl]
                if elu:
                    v = jnp.where(v > 0, v, jnp.exp(jnp.minimum(v, 0.0)) - 1.0)
                r0[r, sl] = v
        return carry

    def _copyout(cnt, i, carry):
        b0 = base + i * K
        pltpu.sync_copy(acc_s.at[pl.ds(b0, cnt)], r0.at[pl.ds(0, cnt)])
        pltpu.sync_copy(den_s.at[pl.ds(b0, cnt)], den_b.at[pl.ds(0, cnt)])
        lax.fori_loop(0, (cnt + 15) // 16, _norm, 0)
        pltpu.sync_copy(r0.at[pl.ds(0, cnt)], out.at[cid, pl.ds(b0, cnt)])
        return carry

    lax.fori_loop(0, nfull, functools.partial(_copyout, K), 0)
    _copyout(rem, nfull, 0)


def _make_sc(elu):
    mesh = plsc.VectorSubcoreMesh(core_axis_name="c", subcore_axis_name="s")
    return pl.kernel(
        functools.partial(_sc_body, elu),
        out_type=jax.ShapeDtypeStruct((2, NP, HALF), jnp.float32),
        mesh=mesh,
        compiler_params=pltpu.CompilerParams(needs_layout_passes=False),
        scratch_types=[
            pltpu.VMEM((HM, K), jnp.int32),
            pltpu.VMEM((HM, K), jnp.int32),
            pltpu.VMEM((K, HALF), jnp.float32),
            pltpu.VMEM((K, HALF), jnp.float32),
            pltpu.VMEM((K,), jnp.float32),
            pltpu.VMEM((K,), jnp.float32),
            pltpu.VMEM((K,), jnp.float32),
            pltpu.VMEM((K,), jnp.float32),
            pltpu.VMEM((HALF,), jnp.float32),
            pltpu.VMEM((640,), jnp.float32),
            pltpu.VMEM((K,), jnp.int32),
            pltpu.VMEM((K,), jnp.float32),
            pltpu.VMEM((K,), jnp.float32),
            pltpu.VMEM((K,), jnp.float32),
            pltpu.VMEM((K,), jnp.int32),
            pltpu.VMEM_SHARED((NP,), jnp.float32),
            pltpu.VMEM_SHARED((NP,), jnp.float32),
            pltpu.VMEM_SHARED((NP,), jnp.float32),
            pltpu.VMEM_SHARED((NP, HALF), jnp.float32),
            pltpu.SemaphoreType.DMA,
            pltpu.SemaphoreType.DMA,
            pltpu.SemaphoreType.DMA,
            pltpu.SemaphoreType.DMA,
            pltpu.SemaphoreType.DMA,
            pltpu.SemaphoreType.DMA,
            pltpu.SemaphoreType.DMA,
            pltpu.SemaphoreType.DMA,
        ],
    )


_sc_layer1 = _make_sc(True)
_sc_layer2 = _make_sc(False)


def _pack_attn(attn_l, attn_r):
    a = jnp.zeros((DIM, 8), jnp.float32)
    a = a.at[:, 0].set(attn_l)
    a = a.at[:, 1].set(attn_r)
    return a


@jax.jit
def kernel(features, edge_index, W1, attn_l1, attn_r1, b1, W2, attn_l2, attn_r2, b2):
    src = edge_index[0]
    dst = edge_index[1]
    pad = EP - E
    srcw = jnp.concatenate([src, jnp.zeros((pad,), jnp.int32)]).reshape(16, NCH, K)
    dstw = jnp.concatenate([dst, jnp.full((pad,), N, jnp.int32)]).reshape(16, NCH, K)

    f2, s = _mm1(features, W1, _pack_attn(attn_l1, attn_r1))
    b1s = jnp.stack([b1[:HALF], b1[HALF:]])
    zpad = jnp.zeros((NP - N,), jnp.float32)
    el1 = jnp.concatenate([s[:, 0], zpad])
    er1 = jnp.concatenate([s[:, 1], zpad])
    h = _sc_layer1(f2[0], f2[1], el1, er1, srcw, dstw, b1s)

    f2b, s2 = _mm2(h[0], h[1], W2[:HALF], W2[HALF:], _pack_attn(attn_l2, attn_r2))
    b2s = jnp.stack([b2[:HALF], b2[HALF:]])
    o = _sc_layer2(f2b[0], f2b[1], s2[:, 0], s2[:, 1], srcw, dstw, b2s)
    return jnp.concatenate([o[0, :N], o[1, :N]], axis=1)


# speedup vs baseline: 1.0359x; 1.0346x over previous
import functools

import jax
import jax.numpy as jnp
from jax import lax
from jax.experimental import pallas as pl
from jax.experimental.pallas import tpu as pltpu
from jax.experimental.pallas import tpu_sc as plsc

N = 10000
NP = 10112
SP = NP // 16
E = 160000
K = 128
HM = 40
NMEGA = 2
NCH = HM * NMEGA
EP = 16 * NCH * K
DIM = 256
HALF = 128
BN1 = 1000
BN2 = 632


def _mm1_body(x_ref, w_ref, a_ref, f_ref, s_ref):
    feat = jnp.dot(x_ref[...], w_ref[...], preferred_element_type=jnp.float32)
    f_ref[0] = feat[:, :HALF]
    f_ref[1] = feat[:, HALF:]
    s_ref[...] = jnp.dot(feat, a_ref[...], preferred_element_type=jnp.float32)


def _mm1(x, w, a):
    return pl.pallas_call(
        _mm1_body,
        grid=(N // BN1,),
        in_specs=[
            pl.BlockSpec((BN1, DIM), lambda i: (i, 0)),
            pl.BlockSpec((DIM, DIM), lambda i: (0, 0)),
            pl.BlockSpec((DIM, 8), lambda i: (0, 0)),
        ],
        out_specs=[
            pl.BlockSpec((2, BN1, HALF), lambda i: (0, i, 0)),
            pl.BlockSpec((BN1, 8), lambda i: (i, 0)),
        ],
        out_shape=[
            jax.ShapeDtypeStruct((2, N, HALF), jnp.float32),
            jax.ShapeDtypeStruct((N, 8), jnp.float32),
        ],
    )(x, w, a)


def _mm2_body(xa_ref, xb_ref, wa_ref, wb_ref, a_ref, f_ref, s_ref):
    feat = jnp.dot(xa_ref[...], wa_ref[...], preferred_element_type=jnp.float32)
    feat += jnp.dot(xb_ref[...], wb_ref[...], preferred_element_type=jnp.float32)
    f_ref[0] = feat[:, :HALF]
    f_ref[1] = feat[:, HALF:]
    s_ref[...] = jnp.dot(feat, a_ref[...], preferred_element_type=jnp.float32)


def _mm2(xa, xb, wa, wb, a):
    return pl.pallas_call(
        _mm2_body,
        grid=(NP // BN2,),
        in_specs=[
            pl.BlockSpec((BN2, HALF), lambda i: (i, 0)),
            pl.BlockSpec((BN2, HALF), lambda i: (i, 0)),
            pl.BlockSpec((HALF, DIM), lambda i: (0, 0)),
            pl.BlockSpec((HALF, DIM), lambda i: (0, 0)),
            pl.BlockSpec((DIM, 8), lambda i: (0, 0)),
        ],
        out_specs=[
            pl.BlockSpec((2, BN2, HALF), lambda i: (0, i, 0)),
            pl.BlockSpec((BN2, 8), lambda i: (i, 0)),
        ],
        out_shape=[
            jax.ShapeDtypeStruct((2, NP, HALF), jnp.float32),
            jax.ShapeDtypeStruct((NP, 8), jnp.float32),
        ],
    )(xa, xb, wa, wb, a)


def _sc_body(elu, tabA, tabB, elh, erh, srcw, dstw, bias2, out,
             src_v, dst_v, r0, r1, els_b, erd_b, ee_b, den_b, bias_v, zv, idx_b,
             el_s, er_s, den_s, acc_s, sem0, sem1):
    cid = lax.axis_index("c")
    sid = lax.axis_index("s")
    zero16 = jnp.zeros((16,), jnp.float32)
    base = sid * SP

    @pl.when(sid == 0)
    def _():
        pltpu.sync_copy(elh, el_s)

    @pl.when(sid == 1)
    def _():
        pltpu.sync_copy(erh, er_s)

    pltpu.sync_copy(bias2.at[cid], bias_v)

    def _z_row(r, carry):
        for j in range(8):
            r0[r, pl.ds(j * 16, 16)] = zero16
        return carry
    lax.fori_loop(0, K, _z_row, 0)
    for i in range(40):
        zv[pl.ds(i * 16, 16)] = zero16
    nfull = SP // K
    rem = SP - nfull * K
    for i in range(nfull):
        pltpu.sync_copy(r0, acc_s.at[pl.ds(base + i * K, K)])
    pltpu.sync_copy(r0.at[pl.ds(0, rem)], acc_s.at[pl.ds(base + nfull * K, rem)])
    pltpu.sync_copy(zv.at[pl.ds(0, SP)], den_s.at[pl.ds(base, SP)])
    plsc.subcore_barrier()

    rows = (r0, r1)
    sems = (sem0, sem1)

    def _start_gather(c, buf, sem):
        @pl.when(cid == 0)
        def _():
            pltpu.async_copy(tabA.at[src_v.at[c]], buf, sem)

        @pl.when(cid == 1)
        def _():
            pltpu.async_copy(tabB.at[src_v.at[c]], buf, sem)

    for m in range(NMEGA):
        pltpu.sync_copy(srcw.at[sid, pl.ds(m * HM, HM)], src_v)
        pltpu.sync_copy(dstw.at[sid, pl.ds(m * HM, HM)], dst_v)
        _start_gather(0, r0, sem0)
        _start_gather(1, r1, sem1)

        def _chunk(g, carry):
            for b in range(2):
                c = g * 2 + b
                pltpu.sync_copy(el_s.at[src_v.at[c]], els_b)
                pltpu.sync_copy(er_s.at[dst_v.at[c]], erd_b)
                for j in range(K // 16):
                    sl = pl.ds(j * 16, 16)
                    e = els_b[sl] + erd_b[sl]
                    e = jnp.where(e > 0, e, 0.2 * e)
                    ee = jnp.exp(e)
                    ee_b[sl] = jnp.where(dst_v[c, sl] >= N, 0.0, ee)
                for j in range(K // 16):
                    sl = pl.ds(j * 16, 16)
                    idx_b[sl] = dst_v[c, sl]
                pltpu.sync_copy(ee_b, den_s.at[idx_b], add=True)

                pltpu.make_async_copy(
                    tabA.at[src_v.at[c]], rows[b], sems[b]).wait()

                @plsc.parallel_loop(0, K // 16, unroll=2)
                def _scale(g2):
                    a16 = ee_b[pl.ds(g2 * 16, 16)]
                    for i in range(16):
                        r = g2 * 16 + i
                        av = jnp.full((16,), a16[i], jnp.float32)
                        for j in range(8):
                            sl = pl.ds(j * 16, 16)
                            rows[b][r, sl] = rows[b][r, sl] * av
                pltpu.sync_copy(rows[b], acc_s.at[idx_b], add=True)
                nc = c + 2

                @pl.when(nc < HM)
                def _():
                    _start_gather(nc, rows[b], sems[b])
            return carry
        lax.fori_loop(0, HM // 2, _chunk, 0)
    plsc.subcore_barrier()

    def _norm_g(g2):
        d16 = jnp.maximum(den_b[pl.ds(g2 * 16, 16)], 1e-9)
        inv16 = 1.0 / d16
        for i2 in range(16):
            r = g2 * 16 + i2
            dv = jnp.full((16,), inv16[i2], jnp.float32)
            for j in range(8):
                sl = pl.ds(j * 16, 16)
                v = r0[r, sl] * dv + bias_v[sl]
                if elu:
                    v = jnp.where(v > 0, v, jnp.exp(jnp.minimum(v, 0.0)) - 1.0)
                r0[r, sl] = v

    def _copyout(cnt, i, carry):
        b0 = base + i * K
        pltpu.sync_copy(acc_s.at[pl.ds(b0, cnt)], r0.at[pl.ds(0, cnt)])
        pltpu.sync_copy(den_s.at[pl.ds(b0, cnt)], den_b.at[pl.ds(0, cnt)])
        plsc.parallel_loop(0, (cnt + 15) // 16, unroll=2)(_norm_g)
        pltpu.sync_copy(r0.at[pl.ds(0, cnt)], out.at[cid, pl.ds(b0, cnt)])
        return carry

    lax.fori_loop(0, nfull, functools.partial(_copyout, K), 0)
    _copyout(rem, nfull, 0)


def _make_sc(elu):
    mesh = plsc.VectorSubcoreMesh(core_axis_name="c", subcore_axis_name="s")
    return pl.kernel(
        functools.partial(_sc_body, elu),
        out_type=jax.ShapeDtypeStruct((2, NP, HALF), jnp.float32),
        mesh=mesh,
        compiler_params=pltpu.CompilerParams(needs_layout_passes=False),
        scratch_types=[
            pltpu.VMEM((HM, K), jnp.int32),
            pltpu.VMEM((HM, K), jnp.int32),
            pltpu.VMEM((K, HALF), jnp.float32),
            pltpu.VMEM((K, HALF), jnp.float32),
            pltpu.VMEM((K,), jnp.float32),
            pltpu.VMEM((K,), jnp.float32),
            pltpu.VMEM((K,), jnp.float32),
            pltpu.VMEM((K,), jnp.float32),
            pltpu.VMEM((HALF,), jnp.float32),
            pltpu.VMEM((640,), jnp.float32),
            pltpu.VMEM((K,), jnp.int32),
            pltpu.VMEM_SHARED((NP,), jnp.float32),
            pltpu.VMEM_SHARED((NP,), jnp.float32),
            pltpu.VMEM_SHARED((NP,), jnp.float32),
            pltpu.VMEM_SHARED((NP, HALF), jnp.float32),
            pltpu.SemaphoreType.DMA,
            pltpu.SemaphoreType.DMA,
        ],
    )


_sc_layer1 = _make_sc(True)
_sc_layer2 = _make_sc(False)


def _pack_attn(attn_l, attn_r):
    a = jnp.zeros((DIM, 8), jnp.float32)
    a = a.at[:, 0].set(attn_l)
    a = a.at[:, 1].set(attn_r)
    return a


@jax.jit
def kernel(features, edge_index, W1, attn_l1, attn_r1, b1, W2, attn_l2, attn_r2, b2):
    src = edge_index[0]
    dst = edge_index[1]
    pad = EP - E
    srcw = jnp.concatenate([src, jnp.zeros((pad,), jnp.int32)]).reshape(16, NCH, K)
    dstw = jnp.concatenate([dst, jnp.full((pad,), N, jnp.int32)]).reshape(16, NCH, K)

    f2, s = _mm1(features, W1, _pack_attn(attn_l1, attn_r1))
    b1s = jnp.stack([b1[:HALF], b1[HALF:]])
    zpad = jnp.zeros((NP - N,), jnp.float32)
    el1 = jnp.concatenate([s[:, 0], zpad])
    er1 = jnp.concatenate([s[:, 1], zpad])
    h = _sc_layer1(f2[0], f2[1], el1, er1, srcw, dstw, b1s)

    f2b, s2 = _mm2(h[0], h[1], W2[:HALF], W2[HALF:], _pack_attn(attn_l2, attn_r2))
    b2s = jnp.stack([b2[:HALF], b2[HALF:]])
    o = _sc_layer2(f2b[0], f2b[1], s2[:, 0], s2[:, 1], srcw, dstw, b2s)
    return jnp.concatenate([o[0, :N], o[1, :N]], axis=1)


# speedup vs baseline: 1.0814x; 1.0439x over previous
import functools

import jax
import jax.numpy as jnp
from jax import lax
from jax.experimental import pallas as pl
from jax.experimental.pallas import tpu as pltpu
from jax.experimental.pallas import tpu_sc as plsc

N = 10000
NP = 10112
SP = NP // 16
E = 160000
K = 128
HM = 40
NMEGA = 2
NCH = HM * NMEGA
EP = 16 * NCH * K
DIM = 256
HALF = 128
BN1 = 1000
BN2 = 632


def _mm1_body(x_ref, w_ref, a_ref, f_ref, s_ref):
    feat = jnp.dot(x_ref[...], w_ref[...], preferred_element_type=jnp.float32)
    f_ref[0] = feat[:, :HALF]
    f_ref[1] = feat[:, HALF:]
    s_ref[...] = jnp.dot(feat, a_ref[...], preferred_element_type=jnp.float32)


def _mm1(x, w, a):
    return pl.pallas_call(
        _mm1_body,
        grid=(N // BN1,),
        in_specs=[
            pl.BlockSpec((BN1, DIM), lambda i: (i, 0)),
            pl.BlockSpec((DIM, DIM), lambda i: (0, 0)),
            pl.BlockSpec((DIM, 8), lambda i: (0, 0)),
        ],
        out_specs=[
            pl.BlockSpec((2, BN1, HALF), lambda i: (0, i, 0)),
            pl.BlockSpec((BN1, 8), lambda i: (i, 0)),
        ],
        out_shape=[
            jax.ShapeDtypeStruct((2, N, HALF), jnp.float32),
            jax.ShapeDtypeStruct((N, 8), jnp.float32),
        ],
    )(x, w, a)


def _mm2_body(xa_ref, xb_ref, wa_ref, wb_ref, a_ref, f_ref, s_ref):
    feat = jnp.dot(xa_ref[...], wa_ref[...], preferred_element_type=jnp.float32)
    feat += jnp.dot(xb_ref[...], wb_ref[...], preferred_element_type=jnp.float32)
    f_ref[0] = feat[:, :HALF]
    f_ref[1] = feat[:, HALF:]
    s_ref[...] = jnp.dot(feat, a_ref[...], preferred_element_type=jnp.float32)


def _mm2(xa, xb, wa, wb, a):
    return pl.pallas_call(
        _mm2_body,
        grid=(NP // BN2,),
        in_specs=[
            pl.BlockSpec((BN2, HALF), lambda i: (i, 0)),
            pl.BlockSpec((BN2, HALF), lambda i: (i, 0)),
            pl.BlockSpec((HALF, DIM), lambda i: (0, 0)),
            pl.BlockSpec((HALF, DIM), lambda i: (0, 0)),
            pl.BlockSpec((DIM, 8), lambda i: (0, 0)),
        ],
        out_specs=[
            pl.BlockSpec((2, BN2, HALF), lambda i: (0, i, 0)),
            pl.BlockSpec((BN2, 8), lambda i: (i, 0)),
        ],
        out_shape=[
            jax.ShapeDtypeStruct((2, NP, HALF), jnp.float32),
            jax.ShapeDtypeStruct((NP, 8), jnp.float32),
        ],
    )(xa, xb, wa, wb, a)


def _sc_body(elu, tabA, tabB, elh, erh, srcw, dstw, bias2, out,
             src_v, dst_v, r0, r1, els_b, erd_b, ee_b, den_b, bias_v, zv, idx_b,
             el_s, er_s, den_s, acc_s, sem0, sem1):
    cid = lax.axis_index("c")
    sid = lax.axis_index("s")
    zero16 = jnp.zeros((16,), jnp.float32)
    base = sid * SP

    @pl.when(sid == 0)
    def _():
        pltpu.sync_copy(elh, el_s)

    @pl.when(sid == 1)
    def _():
        pltpu.sync_copy(erh, er_s)

    pltpu.sync_copy(bias2.at[cid], bias_v)

    def _z_row(r, carry):
        for j in range(8):
            r0[r, pl.ds(j * 16, 16)] = zero16
        return carry
    lax.fori_loop(0, K, _z_row, 0)
    for i in range(40):
        zv[pl.ds(i * 16, 16)] = zero16
    nfull = SP // K
    rem = SP - nfull * K
    for i in range(nfull):
        pltpu.sync_copy(r0, acc_s.at[pl.ds(base + i * K, K)])
    pltpu.sync_copy(r0.at[pl.ds(0, rem)], acc_s.at[pl.ds(base + nfull * K, rem)])
    pltpu.sync_copy(zv.at[pl.ds(0, SP)], den_s.at[pl.ds(base, SP)])
    plsc.subcore_barrier()

    rows = (r0, r1)
    sems = (sem0, sem1)

    def _start_gather(c, buf, sem):
        @pl.when(cid == 0)
        def _():
            pltpu.async_copy(tabA.at[src_v.at[c]], buf, sem)

        @pl.when(cid == 1)
        def _():
            pltpu.async_copy(tabB.at[src_v.at[c]], buf, sem)

    for m in range(NMEGA):
        pltpu.sync_copy(srcw.at[sid, pl.ds(m * HM, HM)], src_v)
        pltpu.sync_copy(dstw.at[sid, pl.ds(m * HM, HM)], dst_v)
        _start_gather(0, r0, sem0)
        _start_gather(1, r1, sem1)

        def _chunk(g, carry):
            for b in range(2):
                c = g * 2 + b
                pltpu.sync_copy(el_s.at[src_v.at[c]], els_b)
                pltpu.sync_copy(er_s.at[dst_v.at[c]], erd_b)
                for j in range(K // 16):
                    sl = pl.ds(j * 16, 16)
                    e = els_b[sl] + erd_b[sl]
                    e = jnp.where(e > 0, e, 0.2 * e)
                    ee = jnp.exp(e)
                    ee_b[sl] = jnp.where(dst_v[c, sl] >= N, 0.0, ee)
                for j in range(K // 16):
                    sl = pl.ds(j * 16, 16)
                    idx_b[sl] = dst_v[c, sl]
                pltpu.sync_copy(ee_b, den_s.at[idx_b], add=True)

                pltpu.make_async_copy(
                    tabA.at[src_v.at[c]], rows[b], sems[b]).wait()

                def _scale(g2, carry2):
                    a16 = ee_b[pl.ds(g2 * 16, 16)]
                    for i in range(16):
                        r = g2 * 16 + i
                        av = jnp.full((16,), a16[i], jnp.float32)
                        for j in range(8):
                            sl = pl.ds(j * 16, 16)
                            rows[b][r, sl] = rows[b][r, sl] * av
                    return carry2
                lax.fori_loop(0, K // 16, _scale, 0)
                pltpu.sync_copy(rows[b], acc_s.at[idx_b], add=True)
                nc = c + 2

                @pl.when(nc < HM)
                def _():
                    _start_gather(nc, rows[b], sems[b])
            return carry
        lax.fori_loop(0, HM // 2, _chunk, 0)
    plsc.subcore_barrier()

    def _norm(g2, carry):
        d16 = jnp.maximum(den_b[pl.ds(g2 * 16, 16)], 1e-9)
        inv16 = 1.0 / d16
        for i2 in range(16):
            r = g2 * 16 + i2
            dv = jnp.full((16,), inv16[i2], jnp.float32)
            for j in range(8):
                sl = pl.ds(j * 16, 16)
                v = r0[r, sl] * dv + bias_v[sl]
                if elu:
                    v = jnp.where(v > 0, v, jnp.exp(jnp.minimum(v, 0.0)) - 1.0)
                r0[r, sl] = v
        return carry

    def _copyout(cnt, i, carry):
        b0 = base + i * K
        pltpu.sync_copy(acc_s.at[pl.ds(b0, cnt)], r0.at[pl.ds(0, cnt)])
        pltpu.sync_copy(den_s.at[pl.ds(b0, cnt)], den_b.at[pl.ds(0, cnt)])
        lax.fori_loop(0, (cnt + 15) // 16, _norm, 0)
        pltpu.sync_copy(r0.at[pl.ds(0, cnt)], out.at[cid, pl.ds(b0, cnt)])
        return carry

    lax.fori_loop(0, nfull, functools.partial(_copyout, K), 0)
    _copyout(rem, nfull, 0)


def _make_sc(elu):
    mesh = plsc.VectorSubcoreMesh(core_axis_name="c", subcore_axis_name="s")
    return pl.kernel(
        functools.partial(_sc_body, elu),
        out_type=jax.ShapeDtypeStruct((2, NP, HALF), jnp.float32),
        mesh=mesh,
        compiler_params=pltpu.CompilerParams(needs_layout_passes=False),
        scratch_types=[
            pltpu.VMEM((HM, K), jnp.int32),
            pltpu.VMEM((HM, K), jnp.int32),
            pltpu.VMEM((K, HALF), jnp.float32),
            pltpu.VMEM((K, HALF), jnp.float32),
            pltpu.VMEM((K,), jnp.float32),
            pltpu.VMEM((K,), jnp.float32),
            pltpu.VMEM((K,), jnp.float32),
            pltpu.VMEM((K,), jnp.float32),
            pltpu.VMEM((HALF,), jnp.float32),
            pltpu.VMEM((640,), jnp.float32),
            pltpu.VMEM((K,), jnp.int32),
            pltpu.VMEM_SHARED((NP,), jnp.float32),
            pltpu.VMEM_SHARED((NP,), jnp.float32),
            pltpu.VMEM_SHARED((NP,), jnp.float32),
            pltpu.VMEM_SHARED((NP, HALF), jnp.float32),
            pltpu.SemaphoreType.DMA,
            pltpu.SemaphoreType.DMA,
        ],
    )


_sc_layer1 = _make_sc(True)
_sc_layer2 = _make_sc(False)


def _pack_attn(attn_l, attn_r):
    a = jnp.zeros((DIM, 8), jnp.float32)
    a = a.at[:, 0].set(attn_l)
    a = a.at[:, 1].set(attn_r)
    return a


@jax.jit
def kernel(features, edge_index, W1, attn_l1, attn_r1, b1, W2, attn_l2, attn_r2, b2):
    src = edge_index[0]
    dst = edge_index[1]
    pad = EP - E
    srcw = jnp.concatenate([src, jnp.zeros((pad,), jnp.int32)]).reshape(16, NCH, K)
    dstw = jnp.concatenate([dst, jnp.full((pad,), N, jnp.int32)]).reshape(16, NCH, K)

    f2, s = _mm1(features, W1, _pack_attn(attn_l1, attn_r1))
    b1s = jnp.stack([b1[:HALF], b1[HALF:]])
    zpad = jnp.zeros((NP - N,), jnp.float32)
    el1 = jnp.concatenate([s[:, 0], zpad])
    er1 = jnp.concatenate([s[:, 1], zpad])
    h = _sc_layer1(f2[0], f2[1], el1, er1, srcw, dstw, b1s)

    f2b, s2 = _mm2(h[0], h[1], W2[:HALF], W2[HALF:], _pack_attn(attn_l2, attn_r2))
    b2s = jnp.stack([b2[:HALF], b2[HALF:]])
    o = _sc_layer2(f2b[0], f2b[1], s2[:, 0], s2[:, 1], srcw, dstw, b2s)
    return jnp.concatenate([o[0, :N], o[1, :N]], axis=1)
